# baseline (speedup 1.0000x reference)
import jax
import jax.numpy as jnp
from jax.experimental import pallas as pl
from jax.experimental.pallas import tpu as pltpu


def kernel(x, A, W, b, gamma, beta):
    del b
    n, d_in = x.shape
    d_out = W.shape[1]
    BM = 200
    M = n // BM
    eps = 1e-5

    def body(x_ref, a_ref, w_ref, g_ref, bta_ref, out_ref, h_ref, s_ref):
        g = pl.program_id(0)

        @pl.when(g == 0)
        def _():
            s_ref[...] = jnp.zeros_like(s_ref)

        @pl.when(g < M)
        def _():
            ax = jnp.dot(a_ref[...], x_ref[...],
                         preferred_element_type=jnp.float32)
            h = jnp.dot(ax, w_ref[...], preferred_element_type=jnp.float32,
                        precision=jax.lax.Precision.HIGHEST)
            h_ref[pl.ds(g * BM, BM), :] = h
            s_ref[0:1, :] += jnp.sum(h, axis=0, keepdims=True)
            s_ref[1:2, :] += jnp.sum(h * h, axis=0, keepdims=True)

        @pl.when(g == M)
        def _():
            h = h_ref[...]
            mean = s_ref[0:1, :] * (1.0 / n)
            var = s_ref[1:2, :] * (1.0 / n) - mean * mean
            scale = jax.lax.rsqrt(var + eps) * g_ref[...]
            y = (h - mean) * scale + bta_ref[...]
            out_ref[...] = jnp.where(y >= 0, y, 0.2 * y)

    out = pl.pallas_call(
        body,
        grid=(M + 1,),
        in_specs=[
            pl.BlockSpec((n, d_in), lambda g: (0, 0)),
            pl.BlockSpec((BM, n),
                         lambda g: (jnp.minimum(g, M - 1), 0)),
            pl.BlockSpec((d_in, d_out), lambda g: (0, 0)),
            pl.BlockSpec((1, d_out), lambda g: (0, 0)),
            pl.BlockSpec((1, d_out), lambda g: (0, 0)),
        ],
        out_specs=pl.BlockSpec((n, d_out), lambda g: (0, 0)),
        out_shape=jax.ShapeDtypeStruct((n, d_out), jnp.float32),
        scratch_shapes=[
            pltpu.VMEM((n, d_out), jnp.float32),
            pltpu.VMEM((8, d_out), jnp.float32),
        ],
        compiler_params=pltpu.CompilerParams(
            dimension_semantics=("arbitrary",),
            vmem_limit_bytes=112 * 1024 * 1024,
        ),
    )(x, A, W, gamma.reshape(1, -1), beta.reshape(1, -1))
    return out


# speedup vs baseline: 1.0825x; 1.0825x over previous
import jax
import jax.numpy as jnp
from jax.experimental import pallas as pl
from jax.experimental.pallas import tpu as pltpu


def kernel(x, A, W, b, gamma, beta):
    del b
    n, d_in = x.shape
    d_out = W.shape[1]
    BM = 400
    M = n // BM
    eps = 1e-5

    def body(x_ref, a_ref, w_ref, g_ref, bta_ref, out_ref, h_ref, s_ref):
        g = pl.program_id(0)

        @pl.when(g == 0)
        def _():
            s_ref[...] = jnp.zeros_like(s_ref)

        @pl.when(g < M)
        def _():
            ax = jnp.dot(a_ref[...], x_ref[...],
                         preferred_element_type=jnp.float32)
            h = jnp.dot(ax, w_ref[...], preferred_element_type=jnp.float32,
                        precision=jax.lax.Precision.HIGHEST)
            h_ref[pl.ds(g * BM, BM), :] = h
            s_ref[0:1, :] += jnp.sum(h, axis=0, keepdims=True)
            s_ref[1:2, :] += jnp.sum(h * h, axis=0, keepdims=True)

        @pl.when(g == M)
        def _():
            h = h_ref[...]
            mean = s_ref[0:1, :] * (1.0 / n)
            var = s_ref[1:2, :] * (1.0 / n) - mean * mean
            scale = jax.lax.rsqrt(var + eps) * g_ref[...]
            y = (h - mean) * scale + bta_ref[...]
            out_ref[...] = jnp.where(y >= 0, y, 0.2 * y)

    out = pl.pallas_call(
        body,
        grid=(M + 1,),
        in_specs=[
            pl.BlockSpec((n, d_in), lambda g: (0, 0)),
            pl.BlockSpec((BM, n),
                         lambda g: (jnp.minimum(g, M - 1), 0)),
            pl.BlockSpec((d_in, d_out), lambda g: (0, 0)),
            pl.BlockSpec((1, d_out), lambda g: (0, 0)),
            pl.BlockSpec((1, d_out), lambda g: (0, 0)),
        ],
        out_specs=pl.BlockSpec((n, d_out), lambda g: (0, 0)),
        out_shape=jax.ShapeDtypeStruct((n, d_out), jnp.float32),
        scratch_shapes=[
            pltpu.VMEM((n, d_out), jnp.float32),
            pltpu.VMEM((8, d_out), jnp.float32),
        ],
        compiler_params=pltpu.CompilerParams(
            dimension_semantics=("arbitrary",),
            vmem_limit_bytes=112 * 1024 * 1024,
        ),
    )(x, A, W, gamma.reshape(1, -1), beta.reshape(1, -1))
    return out


# speedup vs baseline: 1.0895x; 1.0065x over previous
import jax
import jax.numpy as jnp
from jax.experimental import pallas as pl
from jax.experimental.pallas import tpu as pltpu


def kernel(x, A, W, b, gamma, beta):
    del b
    n, d_in = x.shape
    d_out = W.shape[1]
    BM = 400
    M = n // BM
    eps = 1e-5

    def body(x_ref, a_ref, w_ref, g_ref, bta_ref, out_ref, h_ref, s_ref):
        g = pl.program_id(0)

        @pl.when(g == 0)
        def _():
            s_ref[...] = jnp.zeros_like(s_ref)

        ax = jnp.dot(a_ref[...], x_ref[...],
                     preferred_element_type=jnp.float32)
        h = jnp.dot(ax, w_ref[...], preferred_element_type=jnp.float32,
                    precision=jax.lax.Precision.HIGHEST)
        s_ref[0:1, :] += jnp.sum(h, axis=0, keepdims=True)
        s_ref[1:2, :] += jnp.sum(h * h, axis=0, keepdims=True)

        @pl.when(g < M - 1)
        def _():
            h_ref[pl.ds(g * BM, BM), :] = h

        @pl.when(g == M - 1)
        def _():
            mean = s_ref[0:1, :] * (1.0 / n)
            var = s_ref[1:2, :] * (1.0 / n) - mean * mean
            scale = jax.lax.rsqrt(var + eps) * g_ref[...]
            y0 = (h_ref[...] - mean) * scale + bta_ref[...]
            out_ref[0:(M - 1) * BM, :] = jnp.where(y0 >= 0, y0, 0.2 * y0)
            y1 = (h - mean) * scale + bta_ref[...]
            out_ref[(M - 1) * BM:n, :] = jnp.where(y1 >= 0, y1, 0.2 * y1)

    out = pl.pallas_call(
        body,
        grid=(M,),
        in_specs=[
            pl.BlockSpec((n, d_in), lambda g: (0, 0)),
            pl.BlockSpec((BM, n), lambda g: (g, 0)),
            pl.BlockSpec((d_in, d_out), lambda g: (0, 0)),
            pl.BlockSpec((1, d_out), lambda g: (0, 0)),
            pl.BlockSpec((1, d_out), lambda g: (0, 0)),
        ],
        out_specs=pl.BlockSpec((n, d_out), lambda g: (0, 0)),
        out_shape=jax.ShapeDtypeStruct((n, d_out), jnp.float32),
        scratch_shapes=[
            pltpu.VMEM((n - BM, d_out), jnp.float32),
            pltpu.VMEM((8, d_out), jnp.float32),
        ],
        compiler_params=pltpu.CompilerParams(
            dimension_semantics=("arbitrary",),
            vmem_limit_bytes=112 * 1024 * 1024,
        ),
    )(x, A, W, gamma.reshape(1, -1), beta.reshape(1, -1))
    return out
